# baseline (speedup 1.0000x reference)
import functools

import jax
import jax.numpy as jnp
from jax import lax
from jax.experimental import pallas as pl
from jax.experimental.pallas import tpu as pltpu
from jax.experimental.pallas import tpu_sc as plsc

BATCH = 4
MAX_LEN = 4096
D_MODEL = 1024
NC = 2
NS = 16
LANES = 16
NW = NC * NS
ROWS_PER_W = MAX_LEN // NW
CHUNK = 8
NCHUNK = ROWS_PER_W // CHUNK
ROW_VREGS = D_MODEL // LANES
NSET = 3


def _sc_add(x, emb):
    mesh = plsc.VectorSubcoreMesh(core_axis_name="c", subcore_axis_name="s")

    scratch = (
        [pltpu.VMEM((CHUNK, D_MODEL), jnp.float32) for _ in range(NSET)]
        + [pltpu.VMEM((CHUNK, D_MODEL), jnp.float32)
           for _ in range(NSET * BATCH)]
        + [pltpu.SemaphoreType.DMA for _ in range(NSET)]
        + [pltpu.SemaphoreType.DMA for _ in range(NSET * BATCH)]
        + [pltpu.SemaphoreType.DMA for _ in range(NSET * BATCH)]
    )

    @functools.partial(
        pl.kernel,
        out_type=jax.ShapeDtypeStruct((BATCH, MAX_LEN, D_MODEL), jnp.float32),
        mesh=mesh,
        scratch_types=scratch,
    )
    def body(x_hbm, emb_hbm, out_hbm, *refs):
        p = 0
        ebuf = refs[p:p + NSET]; p += NSET
        xbuf = [refs[p + s * BATCH:p + (s + 1) * BATCH] for s in range(NSET)]
        p += NSET * BATCH
        esem = refs[p:p + NSET]; p += NSET
        xsem = [refs[p + s * BATCH:p + (s + 1) * BATCH] for s in range(NSET)]
        p += NSET * BATCH
        osem = [refs[p + s * BATCH:p + (s + 1) * BATCH] for s in range(NSET)]

        wid = lax.axis_index("s") * NC + lax.axis_index("c")
        base = wid * ROWS_PER_W

        def issue_loads(ci, st):
            r0 = base + ci * CHUNK
            ed = pltpu.async_copy(
                emb_hbm.at[pl.ds(r0, CHUNK)], ebuf[st], esem[st])
            xd = [pltpu.async_copy(
                x_hbm.at[b, pl.ds(r0, CHUNK)], xbuf[st][b], xsem[st][b])
                for b in range(BATCH)]
            return ed, xd

        def issue_store(ci, st, b):
            r0 = base + ci * CHUNK
            return pltpu.async_copy(
                xbuf[st][b], out_hbm.at[b, pl.ds(r0, CHUNK)], osem[st][b])

        load_descs = [None] * NSET
        store_descs = [[None] * BATCH for _ in range(NSET)]
        load_descs[0] = issue_loads(0, 0)

        for ci in range(NCHUNK):
            cur = ci % NSET
            pf = ci + 1
            if pf < NCHUNK:
                st = pf % NSET
                for b in range(BATCH):
                    if store_descs[st][b] is not None:
                        store_descs[st][b].wait()
                        store_descs[st][b] = None
                load_descs[st] = issue_loads(pf, st)
            ed, xd = load_descs[cur]
            ed.wait()
            for b in range(BATCH):
                xd[b].wait()
                xb = xbuf[cur][b]
                eb = ebuf[cur]

                def add_one(i, _xb=xb, _eb=eb):
                    r = lax.shift_right_logical(i, 6)
                    j = lax.bitwise_and(i, ROW_VREGS - 1)
                    s = pl.ds(j * LANES, LANES)
                    plsc.addupdate(_xb.at[r, s], _eb[r, s])

                plsc.parallel_loop(0, CHUNK * ROW_VREGS, 1, unroll=16)(add_one)
                store_descs[cur][b] = issue_store(ci, cur, b)

        for st in range(NSET):
            for b in range(BATCH):
                if store_descs[st][b] is not None:
                    store_descs[st][b].wait()

    return body(x, emb)


def kernel(x, emb):
    return _sc_add(x, emb)


# speedup vs baseline: 1.0455x; 1.0455x over previous
import functools

import jax
import jax.numpy as jnp
from jax import lax
from jax.experimental import pallas as pl
from jax.experimental.pallas import tpu as pltpu
from jax.experimental.pallas import tpu_sc as plsc

BATCH = 4
MAX_LEN = 4096
D_MODEL = 1024
NC = 2
NS = 16
LANES = 16
NW = NC * NS
ROWS_PER_W = MAX_LEN // NW
CHUNK = 8
NCHUNK = ROWS_PER_W // CHUNK
ROW_VREGS = D_MODEL // LANES
NSET = 3


def _sc_add(x, emb):
    mesh = plsc.VectorSubcoreMesh(core_axis_name="c", subcore_axis_name="s")

    scratch = (
        [pltpu.VMEM((CHUNK, D_MODEL), jnp.float32) for _ in range(NSET)]
        + [pltpu.VMEM((CHUNK, D_MODEL), jnp.float32)
           for _ in range(NSET * BATCH)]
        + [pltpu.SemaphoreType.DMA for _ in range(NSET)]
        + [pltpu.SemaphoreType.DMA for _ in range(NSET * BATCH)]
        + [pltpu.SemaphoreType.DMA for _ in range(NSET * BATCH)]
    )

    @functools.partial(
        pl.kernel,
        out_type=jax.ShapeDtypeStruct((BATCH, MAX_LEN, D_MODEL), jnp.float32),
        mesh=mesh,
        scratch_types=scratch,
    )
    def body(x_hbm, emb_hbm, out_hbm, *refs):
        p = 0
        ebuf = refs[p:p + NSET]; p += NSET
        xbuf = [refs[p + s * BATCH:p + (s + 1) * BATCH] for s in range(NSET)]
        p += NSET * BATCH
        esem = refs[p:p + NSET]; p += NSET
        xsem = [refs[p + s * BATCH:p + (s + 1) * BATCH] for s in range(NSET)]
        p += NSET * BATCH
        osem = [refs[p + s * BATCH:p + (s + 1) * BATCH] for s in range(NSET)]

        wid = lax.axis_index("s") * NC + lax.axis_index("c")
        base = wid * ROWS_PER_W

        def issue_loads(ci, st):
            r0 = base + ci * CHUNK
            ed = pltpu.async_copy(
                emb_hbm.at[pl.ds(r0, CHUNK)], ebuf[st], esem[st])
            xd = [pltpu.async_copy(
                x_hbm.at[b, pl.ds(r0, CHUNK)], xbuf[st][b], xsem[st][b])
                for b in range(BATCH)]
            return ed, xd

        def issue_store(ci, st, b):
            r0 = base + ci * CHUNK
            return pltpu.async_copy(
                xbuf[st][b], out_hbm.at[b, pl.ds(r0, CHUNK)], osem[st][b])

        load_descs = [None] * NSET
        store_descs = [[None] * BATCH for _ in range(NSET)]
        load_descs[0] = issue_loads(0, 0)

        for ci in range(NCHUNK):
            cur = ci % NSET
            pf = ci + 1
            if pf < NCHUNK:
                st = pf % NSET
                for b in range(BATCH):
                    if store_descs[st][b] is not None:
                        store_descs[st][b].wait()
                        store_descs[st][b] = None
                load_descs[st] = issue_loads(pf, st)
            ed, xd = load_descs[cur]
            ed.wait()
            for b in range(BATCH):
                xd[b].wait()
            xbs = xbuf[cur]
            eb = ebuf[cur]

            def add_one(i, _xbs=xbs, _eb=eb):
                r = lax.shift_right_logical(i, 6)
                j = lax.bitwise_and(i, ROW_VREGS - 1)
                s = pl.ds(j * LANES, LANES)
                e = _eb[r, s]
                for b in range(BATCH):
                    plsc.addupdate(_xbs[b].at[r, s], e)

            plsc.parallel_loop(0, CHUNK * ROW_VREGS, 1, unroll=8)(add_one)
            for b in range(BATCH):
                store_descs[cur][b] = issue_store(ci, cur, b)

        for st in range(NSET):
            for b in range(BATCH):
                if store_descs[st][b] is not None:
                    store_descs[st][b].wait()

    return body(x, emb)


def kernel(x, emb):
    return _sc_add(x, emb)
